# baseline (speedup 1.0000x reference)
import functools

import jax
import jax.numpy as jnp
from jax import lax
from jax.experimental import pallas as pl
from jax.experimental.pallas import tpu as pltpu
from jax.experimental.pallas import tpu_sc as plsc

_NS = 16
_NC = 2
_CH = 128


def _segment_sums_sc(x_user, x_item, ei_ui, ei_iu):
    N, D = x_user.shape
    E = ei_ui.shape[1]
    ep = -(-E // (_NS * 8 * _CH)) * 8 * _CH
    e_pad = ep * _NS
    n_pad = -(-(N + 1) // (_NS * 8)) * (_NS * 8)
    zr = n_pad // _NS
    orow = (N // _NS) & ~7
    otail = N - _NS * orow
    assert otail % 8 == 0 and _NS * orow % 8 == 0
    nchunk = ep // _CH

    pad = e_pad - E

    def _pad_ei(ei):
        filler = jnp.concatenate(
            [jnp.zeros((1, pad), jnp.int32),
             jnp.full((1, pad), n_pad - 1, jnp.int32)], axis=0)
        return jnp.concatenate([ei.astype(jnp.int32), filler], axis=1)

    ei_ui_p = _pad_ei(ei_ui)
    ei_iu_p = _pad_ei(ei_iu)
    zblk = jnp.zeros((zr, D), jnp.float32)

    mesh = plsc.VectorSubcoreMesh(core_axis_name="c", subcore_axis_name="s")

    def body(xu, xi, e_ui, e_iu, zb, agg_u, agg_i, sidx, didx, rows, acc):
        c = lax.axis_index("c")
        s = lax.axis_index("s")
        pltpu.sync_copy(zb, acc.at[pl.ds(s * zr, zr)])
        plsc.subcore_barrier()

        def run(ei, xsrc):
            base = s * ep

            def chunk(k, carry):
                off = base + k * _CH
                pltpu.sync_copy(ei.at[0, pl.ds(off, _CH)], sidx)
                pltpu.sync_copy(ei.at[1, pl.ds(off, _CH)], didx)
                pltpu.sync_copy(xsrc.at[sidx], rows)
                pltpu.sync_copy(rows, acc.at[didx], add=True)
                return carry

            lax.fori_loop(0, nchunk, chunk, 0)

        @pl.when(c == 0)
        def _():
            run(e_ui, xu)

        @pl.when(c == 1)
        def _():
            run(e_iu, xi)

        plsc.subcore_barrier()

        def copy_out(agg):
            pltpu.sync_copy(acc.at[pl.ds(s * orow, orow)],
                            agg.at[pl.ds(s * orow, orow)])
            if otail:
                @pl.when(s == 0)
                def _():
                    pltpu.sync_copy(acc.at[pl.ds(_NS * orow, otail)],
                                    agg.at[pl.ds(_NS * orow, otail)])

        @pl.when(c == 0)
        def _():
            copy_out(agg_i)

        @pl.when(c == 1)
        def _():
            copy_out(agg_u)

    f = pl.kernel(
        body,
        out_type=[jax.ShapeDtypeStruct((N, D), jnp.float32),
                  jax.ShapeDtypeStruct((N, D), jnp.float32)],
        mesh=mesh,
        scratch_types=[
            pltpu.VMEM((_CH,), jnp.int32),
            pltpu.VMEM((_CH,), jnp.int32),
            pltpu.VMEM((_CH, D), jnp.float32),
            pltpu.VMEM_SHARED((n_pad, D), jnp.float32),
        ],
    )
    return f(x_user, x_item, ei_ui_p, ei_iu_p, zblk)


def _dense_tc(agg_user, agg_item, x_user, x_item,
              wlT_ui, wrT_ui, wlT_iu, wrT_iu, vecs):
    N, D = x_user.shape
    R = 1000
    assert N % R == 0
    grid = (N // R,)

    def body(au, ai, xu, xi, wlui, wrui, wliu, wriu, v, hu, hi):
        vv = v[...]

        def side(agg, x, wl, wr, bl, g, b):
            o = (jnp.dot(agg[...], wl[...], preferred_element_type=jnp.float32)
                 + jnp.dot(x[...], wr[...], preferred_element_type=jnp.float32)
                 + bl)
            mu = jnp.mean(o, axis=-1, keepdims=True)
            var = jnp.mean(jnp.square(o - mu), axis=-1, keepdims=True)
            y = (o - mu) * lax.rsqrt(var + 1e-5) * g + b
            return jnp.maximum(y, 0.0) + x[...]

        hi[...] = side(ai, xi, wlui, wrui, vv[0:1], vv[1:2], vv[2:3])
        hu[...] = side(au, xu, wliu, wriu, vv[3:4], vv[4:5], vv[5:6])

    node = pl.BlockSpec((R, D), lambda i: (i, 0))
    full = pl.BlockSpec((D, D), lambda i: (0, 0))
    vspec = pl.BlockSpec((8, D), lambda i: (0, 0))
    return pl.pallas_call(
        body,
        grid=grid,
        in_specs=[node, node, node, node, full, full, full, full, vspec],
        out_specs=[node, node],
        out_shape=[jax.ShapeDtypeStruct((N, D), jnp.float32),
                   jax.ShapeDtypeStruct((N, D), jnp.float32)],
    )(agg_user, agg_item, x_user, x_item, wlT_ui, wrT_ui, wlT_iu, wrT_iu, vecs)


def kernel(x_user, x_item, edge_index_ui, edge_index_iu,
           Wl_ui, bl_ui, Wr_ui, Wl_iu, bl_iu, Wr_iu,
           g_user, b_user, g_item, b_item):
    agg_user, agg_item = _segment_sums_sc(
        x_user, x_item, edge_index_ui, edge_index_iu)
    zrow = jnp.zeros_like(bl_ui)
    vecs = jnp.stack([bl_ui, g_item, b_item, bl_iu, g_user, b_user, zrow, zrow])
    h_user, h_item = _dense_tc(
        agg_user, agg_item, x_user, x_item,
        Wl_ui.T, Wr_ui.T, Wl_iu.T, Wr_iu.T, vecs)
    return (h_user, h_item)


# speedup vs baseline: 1.5764x; 1.5764x over previous
import functools

import jax
import jax.numpy as jnp
from jax import lax
from jax.experimental import pallas as pl
from jax.experimental.pallas import tpu as pltpu
from jax.experimental.pallas import tpu_sc as plsc

_NS = 16
_NC = 2
_CH = 128


def _segment_sums_sc(x_user, x_item, ei_ui, ei_iu):
    N, D = x_user.shape
    E = ei_ui.shape[1]
    ep = -(-E // (_NS * 8 * _CH)) * 8 * _CH
    e_pad = ep * _NS
    n_pad = -(-(N + 1) // (_NS * 8)) * (_NS * 8)
    zr = n_pad // _NS
    orow = (N // _NS) & ~7
    otail = N - _NS * orow
    assert otail % 8 == 0 and _NS * orow % 8 == 0
    nchunk = ep // _CH

    pad = e_pad - E

    def _pad_ei(ei):
        ar = jnp.arange(pad, dtype=jnp.int32)
        filler = jnp.stack([(ar * 97) % N, N + ar % (n_pad - N)])
        return jnp.concatenate([ei.astype(jnp.int32), filler], axis=1)

    ei_ui_p = _pad_ei(ei_ui)
    ei_iu_p = _pad_ei(ei_iu)
    zblk = jnp.zeros((zr, D), jnp.float32)

    mesh = plsc.VectorSubcoreMesh(core_axis_name="c", subcore_axis_name="s")

    def body(xu, xi, e_ui, e_iu, zb, agg_u, agg_i, sidx, didx, rows, acc):
        c = lax.axis_index("c")
        s = lax.axis_index("s")
        pltpu.sync_copy(zb, acc.at[pl.ds(s * zr, zr)])
        plsc.subcore_barrier()

        def run(ei, xsrc):
            base = s * ep

            def chunk(k, carry):
                off = base + k * _CH
                pltpu.sync_copy(ei.at[0, pl.ds(off, _CH)], sidx)
                pltpu.sync_copy(ei.at[1, pl.ds(off, _CH)], didx)
                pltpu.sync_copy(xsrc.at[sidx], rows)
                pltpu.sync_copy(rows, acc.at[didx], add=True)
                return carry

            lax.fori_loop(0, nchunk, chunk, 0)

        @pl.when(c == 0)
        def _():
            run(e_ui, xu)

        @pl.when(c == 1)
        def _():
            run(e_iu, xi)

        plsc.subcore_barrier()

        def copy_out(agg):
            pltpu.sync_copy(acc.at[pl.ds(s * orow, orow)],
                            agg.at[pl.ds(s * orow, orow)])
            if otail:
                @pl.when(s == 0)
                def _():
                    pltpu.sync_copy(acc.at[pl.ds(_NS * orow, otail)],
                                    agg.at[pl.ds(_NS * orow, otail)])

        @pl.when(c == 0)
        def _():
            copy_out(agg_i)

        @pl.when(c == 1)
        def _():
            copy_out(agg_u)

    f = pl.kernel(
        body,
        out_type=[jax.ShapeDtypeStruct((N, D), jnp.float32),
                  jax.ShapeDtypeStruct((N, D), jnp.float32)],
        mesh=mesh,
        scratch_types=[
            pltpu.VMEM((_CH,), jnp.int32),
            pltpu.VMEM((_CH,), jnp.int32),
            pltpu.VMEM((_CH, D), jnp.float32),
            pltpu.VMEM_SHARED((n_pad, D), jnp.float32),
        ],
    )
    return f(x_user, x_item, ei_ui_p, ei_iu_p, zblk)


def _dense_tc(agg_user, agg_item, x_user, x_item,
              wlT_ui, wrT_ui, wlT_iu, wrT_iu, vecs):
    N, D = x_user.shape
    R = 1000
    assert N % R == 0
    grid = (N // R,)

    def body(au, ai, xu, xi, wlui, wrui, wliu, wriu, v, hu, hi):
        vv = v[...]

        def side(agg, x, wl, wr, bl, g, b):
            o = (jnp.dot(agg[...], wl[...], preferred_element_type=jnp.float32)
                 + jnp.dot(x[...], wr[...], preferred_element_type=jnp.float32)
                 + bl)
            mu = jnp.mean(o, axis=-1, keepdims=True)
            var = jnp.mean(jnp.square(o - mu), axis=-1, keepdims=True)
            y = (o - mu) * lax.rsqrt(var + 1e-5) * g + b
            return jnp.maximum(y, 0.0) + x[...]

        hi[...] = side(ai, xi, wlui, wrui, vv[0:1], vv[1:2], vv[2:3])
        hu[...] = side(au, xu, wliu, wriu, vv[3:4], vv[4:5], vv[5:6])

    node = pl.BlockSpec((R, D), lambda i: (i, 0))
    full = pl.BlockSpec((D, D), lambda i: (0, 0))
    vspec = pl.BlockSpec((8, D), lambda i: (0, 0))
    return pl.pallas_call(
        body,
        grid=grid,
        in_specs=[node, node, node, node, full, full, full, full, vspec],
        out_specs=[node, node],
        out_shape=[jax.ShapeDtypeStruct((N, D), jnp.float32),
                   jax.ShapeDtypeStruct((N, D), jnp.float32)],
    )(agg_user, agg_item, x_user, x_item, wlT_ui, wrT_ui, wlT_iu, wrT_iu, vecs)


def kernel(x_user, x_item, edge_index_ui, edge_index_iu,
           Wl_ui, bl_ui, Wr_ui, Wl_iu, bl_iu, Wr_iu,
           g_user, b_user, g_item, b_item):
    agg_user, agg_item = _segment_sums_sc(
        x_user, x_item, edge_index_ui, edge_index_iu)
    zrow = jnp.zeros_like(bl_ui)
    vecs = jnp.stack([bl_ui, g_item, b_item, bl_iu, g_user, b_user, zrow, zrow])
    h_user, h_item = _dense_tc(
        agg_user, agg_item, x_user, x_item,
        Wl_ui.T, Wr_ui.T, Wl_iu.T, Wr_iu.T, vecs)
    return (h_user, h_item)


# speedup vs baseline: 2.5273x; 1.6033x over previous
import functools

import jax
import jax.numpy as jnp
from jax import lax
from jax.experimental import pallas as pl
from jax.experimental.pallas import tpu as pltpu
from jax.experimental.pallas import tpu_sc as plsc

_NS = 16
_NC = 2
_CH = 128


def _segment_sums_sc(x_user, x_item, ei_ui, ei_iu):
    N, D = x_user.shape
    E = ei_ui.shape[1]
    ep = -(-E // (_NS * 8 * _CH)) * 8 * _CH
    e_pad = ep * _NS
    n_pad = -(-(N + 1) // (_NS * 8)) * (_NS * 8)
    zr = n_pad // _NS
    orow = (N // _NS) & ~7
    otail = N - _NS * orow
    assert otail % 8 == 0 and _NS * orow % 8 == 0
    nchunk = ep // _CH

    pad = e_pad - E

    def _pad_ei(ei):
        ar = jnp.arange(pad, dtype=jnp.int32)
        filler = jnp.stack([(ar * 97) % N, N + ar % (n_pad - N)])
        return jnp.concatenate([ei.astype(jnp.int32), filler], axis=1)

    ei_ui_p = _pad_ei(ei_ui)
    ei_iu_p = _pad_ei(ei_iu)
    zblk = jnp.zeros((zr, D), jnp.float32)

    mesh = plsc.VectorSubcoreMesh(core_axis_name="c", subcore_axis_name="s")

    def body(xu, xi, e_ui, e_iu, zb, agg_u, agg_i,
             sidx0, sidx1, didx0, didx1, rows0, rows1, gsem0, gsem1, acc):
        c = lax.axis_index("c")
        s = lax.axis_index("s")
        pltpu.sync_copy(zb, acc.at[pl.ds(s * zr, zr)])
        plsc.subcore_barrier()
        sidxs = (sidx0, sidx1)
        didxs = (didx0, didx1)
        rows = (rows0, rows1)
        gsems = (gsem0, gsem1)

        def run(ei, xsrc):
            base = s * ep

            def load_idx(k, b):
                off = base + k * _CH
                pltpu.sync_copy(ei.at[0, pl.ds(off, _CH)], sidxs[b])
                pltpu.sync_copy(ei.at[1, pl.ds(off, _CH)], didxs[b])

            load_idx(0, 0)
            pltpu.async_copy(xsrc.at[sidx0], rows0, gsem0)

            def pair(g, carry):
                for b in range(2):
                    k = g * 2 + b

                    @pl.when(k + 1 < nchunk)
                    def _():
                        load_idx(k + 1, 1 - b)
                        pltpu.async_copy(xsrc.at[sidxs[1 - b]],
                                         rows[1 - b], gsems[1 - b])

                    pltpu.make_async_copy(xsrc.at[sidxs[b]],
                                          rows[b], gsems[b]).wait()
                    pltpu.sync_copy(rows[b], acc.at[didxs[b]], add=True)
                return carry

            lax.fori_loop(0, nchunk // 2, pair, 0)

        @pl.when(c == 0)
        def _():
            run(e_ui, xu)

        @pl.when(c == 1)
        def _():
            run(e_iu, xi)

        plsc.subcore_barrier()

        def copy_out(agg):
            pltpu.sync_copy(acc.at[pl.ds(s * orow, orow)],
                            agg.at[pl.ds(s * orow, orow)])
            if otail:
                @pl.when(s == 0)
                def _():
                    pltpu.sync_copy(acc.at[pl.ds(_NS * orow, otail)],
                                    agg.at[pl.ds(_NS * orow, otail)])

        @pl.when(c == 0)
        def _():
            copy_out(agg_i)

        @pl.when(c == 1)
        def _():
            copy_out(agg_u)

    f = pl.kernel(
        body,
        out_type=[jax.ShapeDtypeStruct((N, D), jnp.float32),
                  jax.ShapeDtypeStruct((N, D), jnp.float32)],
        mesh=mesh,
        scratch_types=(
            [pltpu.VMEM((_CH,), jnp.int32)] * 4
            + [pltpu.VMEM((_CH, D), jnp.float32)] * 2
            + [pltpu.SemaphoreType.DMA] * 2
            + [pltpu.VMEM_SHARED((n_pad, D), jnp.float32)]
        ),
    )
    return f(x_user, x_item, ei_ui_p, ei_iu_p, zblk)


def _dense_tc(agg_user, agg_item, x_user, x_item,
              wlT_ui, wrT_ui, wlT_iu, wrT_iu, vecs):
    N, D = x_user.shape
    R = 1000
    assert N % R == 0
    grid = (N // R,)

    def body(au, ai, xu, xi, wlui, wrui, wliu, wriu, v, hu, hi):
        vv = v[...]

        def side(agg, x, wl, wr, bl, g, b):
            o = (jnp.dot(agg[...], wl[...], preferred_element_type=jnp.float32)
                 + jnp.dot(x[...], wr[...], preferred_element_type=jnp.float32)
                 + bl)
            mu = jnp.mean(o, axis=-1, keepdims=True)
            var = jnp.mean(jnp.square(o - mu), axis=-1, keepdims=True)
            y = (o - mu) * lax.rsqrt(var + 1e-5) * g + b
            return jnp.maximum(y, 0.0) + x[...]

        hi[...] = side(ai, xi, wlui, wrui, vv[0:1], vv[1:2], vv[2:3])
        hu[...] = side(au, xu, wliu, wriu, vv[3:4], vv[4:5], vv[5:6])

    node = pl.BlockSpec((R, D), lambda i: (i, 0))
    full = pl.BlockSpec((D, D), lambda i: (0, 0))
    vspec = pl.BlockSpec((8, D), lambda i: (0, 0))
    return pl.pallas_call(
        body,
        grid=grid,
        in_specs=[node, node, node, node, full, full, full, full, vspec],
        out_specs=[node, node],
        out_shape=[jax.ShapeDtypeStruct((N, D), jnp.float32),
                   jax.ShapeDtypeStruct((N, D), jnp.float32)],
    )(agg_user, agg_item, x_user, x_item, wlT_ui, wrT_ui, wlT_iu, wrT_iu, vecs)


def kernel(x_user, x_item, edge_index_ui, edge_index_iu,
           Wl_ui, bl_ui, Wr_ui, Wl_iu, bl_iu, Wr_iu,
           g_user, b_user, g_item, b_item):
    agg_user, agg_item = _segment_sums_sc(
        x_user, x_item, edge_index_ui, edge_index_iu)
    zrow = jnp.zeros_like(bl_ui)
    vecs = jnp.stack([bl_ui, g_item, b_item, bl_iu, g_user, b_user, zrow, zrow])
    h_user, h_item = _dense_tc(
        agg_user, agg_item, x_user, x_item,
        Wl_ui.T, Wr_ui.T, Wl_iu.T, Wr_iu.T, vecs)
    return (h_user, h_item)


# speedup vs baseline: 2.7440x; 1.0857x over previous
import functools

import jax
import jax.numpy as jnp
from jax import lax
from jax.experimental import pallas as pl
from jax.experimental.pallas import tpu as pltpu
from jax.experimental.pallas import tpu_sc as plsc

_NS = 16
_NC = 2
_CH = 128


def _segment_sums_sc(x_user, x_item, ei_ui, ei_iu):
    N, D = x_user.shape
    E = ei_ui.shape[1]
    ep = -(-E // (_NS * 3 * _CH)) * 3 * _CH
    e_pad = ep * _NS
    n_pad = -(-(N + 1) // (_NS * 8)) * (_NS * 8)
    zr = n_pad // _NS
    orow = (N // _NS) & ~7
    otail = N - _NS * orow
    assert otail % 8 == 0 and _NS * orow % 8 == 0
    nchunk = ep // _CH

    pad = e_pad - E

    assert E % _NS == 0 and pad % _NS == 0

    def _pad_ei(ei):
        ar = jnp.arange(pad, dtype=jnp.int32)
        filler = jnp.stack([(ar * 97) % N, N + ar % (n_pad - N)])

        def mix(row, fill):
            return jnp.concatenate(
                [row.reshape(_NS, E // _NS),
                 fill.reshape(_NS, pad // _NS)], axis=1).reshape(-1)

        ei = ei.astype(jnp.int32)
        return jnp.stack([mix(ei[0], filler[0]), mix(ei[1], filler[1])])

    ei_ui_p = _pad_ei(ei_ui)
    ei_iu_p = _pad_ei(ei_iu)
    zblk = jnp.zeros((zr, D), jnp.float32)

    mesh = plsc.VectorSubcoreMesh(core_axis_name="c", subcore_axis_name="s")

    def body(xu, xi, e_ui, e_iu, zb, agg_u, agg_i,
             sidx0, sidx1, sidx2, didx0, didx1, didx2, rows0, rows1, rows2,
             gsem0, gsem1, gsem2, ssem0, ssem1, ssem2, acc):
        c = lax.axis_index("c")
        s = lax.axis_index("s")
        pltpu.sync_copy(zb, acc.at[pl.ds(s * zr, zr)])
        plsc.subcore_barrier()
        sidxs = (sidx0, sidx1, sidx2)
        didxs = (didx0, didx1, didx2)
        rows = (rows0, rows1, rows2)
        gsems = (gsem0, gsem1, gsem2)
        ssems = (ssem0, ssem1, ssem2)

        def run(ei, xsrc):
            base = s * ep

            def load_idx(k, b):
                off = base + k * _CH
                pltpu.sync_copy(ei.at[0, pl.ds(off, _CH)], sidxs[b])
                pltpu.sync_copy(ei.at[1, pl.ds(off, _CH)], didxs[b])

            def wait_scatter(b):
                pltpu.make_async_copy(rows[b], acc.at[didxs[b]],
                                      ssems[b]).wait()

            load_idx(0, 0)
            pltpu.async_copy(xsrc.at[sidx0], rows0, gsem0)

            def trip(g, carry):
                for b in range(3):
                    k = g * 3 + b
                    nb = (b + 1) % 3

                    @pl.when(k >= 2)
                    def _():
                        wait_scatter(nb)

                    @pl.when(k + 1 < nchunk)
                    def _():
                        load_idx(k + 1, nb)
                        pltpu.async_copy(xsrc.at[sidxs[nb]],
                                         rows[nb], gsems[nb])

                    pltpu.make_async_copy(xsrc.at[sidxs[b]],
                                          rows[b], gsems[b]).wait()
                    pltpu.async_copy(rows[b], acc.at[didxs[b]],
                                     ssems[b], add=True)
                return carry

            lax.fori_loop(0, nchunk // 3, trip, 0)
            wait_scatter((nchunk - 2) % 3)
            wait_scatter((nchunk - 1) % 3)

        @pl.when(c == 0)
        def _():
            run(e_ui, xu)

        @pl.when(c == 1)
        def _():
            run(e_iu, xi)

        plsc.subcore_barrier()

        def copy_out(agg):
            pltpu.sync_copy(acc.at[pl.ds(s * orow, orow)],
                            agg.at[pl.ds(s * orow, orow)])
            if otail:
                @pl.when(s == 0)
                def _():
                    pltpu.sync_copy(acc.at[pl.ds(_NS * orow, otail)],
                                    agg.at[pl.ds(_NS * orow, otail)])

        @pl.when(c == 0)
        def _():
            copy_out(agg_i)

        @pl.when(c == 1)
        def _():
            copy_out(agg_u)

    f = pl.kernel(
        body,
        out_type=[jax.ShapeDtypeStruct((N, D), jnp.float32),
                  jax.ShapeDtypeStruct((N, D), jnp.float32)],
        mesh=mesh,
        scratch_types=(
            [pltpu.VMEM((_CH,), jnp.int32)] * 6
            + [pltpu.VMEM((_CH, D), jnp.float32)] * 3
            + [pltpu.SemaphoreType.DMA] * 6
            + [pltpu.VMEM_SHARED((n_pad, D), jnp.float32)]
        ),
    )
    return f(x_user, x_item, ei_ui_p, ei_iu_p, zblk)


def _dense_tc(agg_user, agg_item, x_user, x_item,
              wlT_ui, wrT_ui, wlT_iu, wrT_iu, vecs):
    N, D = x_user.shape
    R = 1000
    assert N % R == 0
    grid = (N // R,)

    def body(au, ai, xu, xi, wlui, wrui, wliu, wriu, v, hu, hi):
        vv = v[...]

        def side(agg, x, wl, wr, bl, g, b):
            o = (jnp.dot(agg[...], wl[...], preferred_element_type=jnp.float32)
                 + jnp.dot(x[...], wr[...], preferred_element_type=jnp.float32)
                 + bl)
            mu = jnp.mean(o, axis=-1, keepdims=True)
            var = jnp.mean(jnp.square(o - mu), axis=-1, keepdims=True)
            y = (o - mu) * lax.rsqrt(var + 1e-5) * g + b
            return jnp.maximum(y, 0.0) + x[...]

        hi[...] = side(ai, xi, wlui, wrui, vv[0:1], vv[1:2], vv[2:3])
        hu[...] = side(au, xu, wliu, wriu, vv[3:4], vv[4:5], vv[5:6])

    node = pl.BlockSpec((R, D), lambda i: (i, 0))
    full = pl.BlockSpec((D, D), lambda i: (0, 0))
    vspec = pl.BlockSpec((8, D), lambda i: (0, 0))
    return pl.pallas_call(
        body,
        grid=grid,
        in_specs=[node, node, node, node, full, full, full, full, vspec],
        out_specs=[node, node],
        out_shape=[jax.ShapeDtypeStruct((N, D), jnp.float32),
                   jax.ShapeDtypeStruct((N, D), jnp.float32)],
    )(agg_user, agg_item, x_user, x_item, wlT_ui, wrT_ui, wlT_iu, wrT_iu, vecs)


def kernel(x_user, x_item, edge_index_ui, edge_index_iu,
           Wl_ui, bl_ui, Wr_ui, Wl_iu, bl_iu, Wr_iu,
           g_user, b_user, g_item, b_item):
    agg_user, agg_item = _segment_sums_sc(
        x_user, x_item, edge_index_ui, edge_index_iu)
    zrow = jnp.zeros_like(bl_ui)
    vecs = jnp.stack([bl_ui, g_item, b_item, bl_iu, g_user, b_user, zrow, zrow])
    h_user, h_item = _dense_tc(
        agg_user, agg_item, x_user, x_item,
        Wl_ui.T, Wr_ui.T, Wl_iu.T, Wr_iu.T, vecs)
    return (h_user, h_item)


# speedup vs baseline: 3.1555x; 1.1500x over previous
import functools

import jax
import jax.numpy as jnp
from jax import lax
from jax.experimental import pallas as pl
from jax.experimental.pallas import tpu as pltpu
from jax.experimental.pallas import tpu_sc as plsc

_NS = 16
_NC = 2
_CH = 128


def _segment_sums_sc(x_user, x_item, ei_ui, ei_iu):
    N, D = x_user.shape
    E = ei_ui.shape[1]
    ep = -(-E // (_NS * 6 * _CH)) * 6 * _CH
    e_pad = ep * _NS
    n_pad = -(-(N + 1) // 8) * 8
    zrm = -(-n_pad // (_NS * 8)) * 8
    zrl = n_pad - (_NS - 1) * zrm
    assert 0 < zrl <= zrm and zrl % 8 == 0
    orow = (N // _NS) & ~7
    otail = N - _NS * orow
    assert otail % 8 == 0 and _NS * orow % 8 == 0
    nchunk = ep // _CH

    pad = e_pad - E

    assert E % _NS == 0 and pad % _NS == 0

    def _pad_ei(ei):
        ar = jnp.arange(pad, dtype=jnp.int32)
        filler = jnp.stack([(ar * 97) % N, N + ar % (n_pad - N)])

        def mix(row, fill):
            return jnp.concatenate(
                [row.reshape(_NS, E // _NS),
                 fill.reshape(_NS, pad // _NS)], axis=1).reshape(-1)

        ei = ei.astype(jnp.int32)
        return jnp.stack([mix(ei[0], filler[0]), mix(ei[1], filler[1])])

    ei_ui_p = _pad_ei(ei_ui)
    ei_iu_p = _pad_ei(ei_iu)
    zblk = jnp.zeros((zrm, D), jnp.float32)

    mesh = plsc.VectorSubcoreMesh(core_axis_name="c", subcore_axis_name="s")

    def body(xu, xi, e_ui, e_iu, zb, agg_u, agg_i,
             idx0, idx1, idx2, idx3, idx4, idx5, rows0, rows1, rows2,
             isem0, isem1, isem2, isem3, isem4, isem5,
             gsem0, gsem1, gsem2, ssem0, ssem1, ssem2, acc):
        c = lax.axis_index("c")
        s = lax.axis_index("s")
        @pl.when(s < _NS - 1)
        def _():
            pltpu.sync_copy(zb, acc.at[pl.ds(s * zrm, zrm)])

        @pl.when(s == _NS - 1)
        def _():
            pltpu.sync_copy(zb.at[pl.ds(0, zrl)],
                            acc.at[pl.ds((_NS - 1) * zrm, zrl)])

        plsc.subcore_barrier()
        idxs = (idx0, idx1, idx2, idx3, idx4, idx5)
        isems = (isem0, isem1, isem2, isem3, isem4, isem5)
        rows = (rows0, rows1, rows2)
        gsems = (gsem0, gsem1, gsem2)
        ssems = (ssem0, ssem1, ssem2)

        def run(ei, xsrc):
            base = s * ep

            def load_idx(k, j):
                pltpu.async_copy(ei.at[:, pl.ds(base + k * _CH, _CH)],
                                 idxs[j], isems[j])

            def wait_idx(k, j):
                pltpu.make_async_copy(ei.at[:, pl.ds(base + k * _CH, _CH)],
                                      idxs[j], isems[j]).wait()

            def wait_scatter(j, b):
                pltpu.make_async_copy(rows[b], acc.at[idxs[j].at[1]],
                                      ssems[b]).wait()

            load_idx(0, 0)
            load_idx(1, 1)
            wait_idx(0, 0)
            pltpu.async_copy(xsrc.at[idx0.at[0]], rows0, gsem0)

            def six(g, carry):
                for b6 in range(6):
                    k = g * 6 + b6
                    b = b6 % 3
                    nb = (b6 + 1) % 3

                    @pl.when(k >= 2)
                    def _():
                        wait_scatter((b6 + 4) % 6, nb)

                    @pl.when(k + 2 < nchunk)
                    def _():
                        load_idx(k + 2, (b6 + 2) % 6)

                    @pl.when(k + 1 < nchunk)
                    def _():
                        wait_idx(k + 1, (b6 + 1) % 6)
                        pltpu.async_copy(xsrc.at[idxs[(b6 + 1) % 6].at[0]],
                                         rows[nb], gsems[nb])

                    pltpu.make_async_copy(xsrc.at[idxs[b6].at[0]],
                                          rows[b], gsems[b]).wait()
                    pltpu.async_copy(rows[b], acc.at[idxs[b6].at[1]],
                                     ssems[b], add=True)
                return carry

            lax.fori_loop(0, nchunk // 6, six, 0)
            wait_scatter((nchunk - 2) % 6, (nchunk - 2) % 3)
            wait_scatter((nchunk - 1) % 6, (nchunk - 1) % 3)

        @pl.when(c == 0)
        def _():
            run(e_ui, xu)

        @pl.when(c == 1)
        def _():
            run(e_iu, xi)

        plsc.subcore_barrier()

        def copy_out(agg):
            pltpu.sync_copy(acc.at[pl.ds(s * orow, orow)],
                            agg.at[pl.ds(s * orow, orow)])
            if otail:
                @pl.when(s == 0)
                def _():
                    pltpu.sync_copy(acc.at[pl.ds(_NS * orow, otail)],
                                    agg.at[pl.ds(_NS * orow, otail)])

        @pl.when(c == 0)
        def _():
            copy_out(agg_i)

        @pl.when(c == 1)
        def _():
            copy_out(agg_u)

    f = pl.kernel(
        body,
        out_type=[jax.ShapeDtypeStruct((N, D), jnp.float32),
                  jax.ShapeDtypeStruct((N, D), jnp.float32)],
        mesh=mesh,
        scratch_types=(
            [pltpu.VMEM((2, _CH), jnp.int32)] * 6
            + [pltpu.VMEM((_CH, D), jnp.float32)] * 3
            + [pltpu.SemaphoreType.DMA] * 12
            + [pltpu.VMEM_SHARED((n_pad, D), jnp.float32)]
        ),
    )
    return f(x_user, x_item, ei_ui_p, ei_iu_p, zblk)


def _dense_tc(agg_user, agg_item, x_user, x_item,
              wlT_ui, wrT_ui, wlT_iu, wrT_iu, vecs):
    N, D = x_user.shape
    R = 1000
    assert N % R == 0
    grid = (N // R,)

    def body(au, ai, xu, xi, wlui, wrui, wliu, wriu, v, hu, hi):
        vv = v[...]

        def side(agg, x, wl, wr, bl, g, b):
            o = (jnp.dot(agg[...], wl[...], preferred_element_type=jnp.float32)
                 + jnp.dot(x[...], wr[...], preferred_element_type=jnp.float32)
                 + bl)
            mu = jnp.mean(o, axis=-1, keepdims=True)
            var = jnp.mean(jnp.square(o - mu), axis=-1, keepdims=True)
            y = (o - mu) * lax.rsqrt(var + 1e-5) * g + b
            return jnp.maximum(y, 0.0) + x[...]

        hi[...] = side(ai, xi, wlui, wrui, vv[0:1], vv[1:2], vv[2:3])
        hu[...] = side(au, xu, wliu, wriu, vv[3:4], vv[4:5], vv[5:6])

    node = pl.BlockSpec((R, D), lambda i: (i, 0))
    full = pl.BlockSpec((D, D), lambda i: (0, 0))
    vspec = pl.BlockSpec((8, D), lambda i: (0, 0))
    return pl.pallas_call(
        body,
        grid=grid,
        in_specs=[node, node, node, node, full, full, full, full, vspec],
        out_specs=[node, node],
        out_shape=[jax.ShapeDtypeStruct((N, D), jnp.float32),
                   jax.ShapeDtypeStruct((N, D), jnp.float32)],
    )(agg_user, agg_item, x_user, x_item, wlT_ui, wrT_ui, wlT_iu, wrT_iu, vecs)


def kernel(x_user, x_item, edge_index_ui, edge_index_iu,
           Wl_ui, bl_ui, Wr_ui, Wl_iu, bl_iu, Wr_iu,
           g_user, b_user, g_item, b_item):
    agg_user, agg_item = _segment_sums_sc(
        x_user, x_item, edge_index_ui, edge_index_iu)
    zrow = jnp.zeros_like(bl_ui)
    vecs = jnp.stack([bl_ui, g_item, b_item, bl_iu, g_user, b_user, zrow, zrow])
    h_user, h_item = _dense_tc(
        agg_user, agg_item, x_user, x_item,
        Wl_ui.T, Wr_ui.T, Wl_iu.T, Wr_iu.T, vecs)
    return (h_user, h_item)
